# baseline (speedup 1.0000x reference)
import jax
import jax.numpy as jnp
from jax.experimental import pallas as pl
from jax.experimental.pallas import tpu as pltpu

H = 128
W = 20
B = 256
HOR = 12
OUTP = 128


def _cell(gi_rz, gi_n, gh, hc):
    r = jax.nn.sigmoid(gi_rz[:, :H] + gh[:, :H])
    z = jax.nn.sigmoid(gi_rz[:, H:] + gh[:, H:2 * H])
    n = jnp.tanh(gi_n + r * gh[:, 2 * H:])
    return (1.0 - z) * n + z * hc


def _body(x0_ref, wr1_ref, bl1_ref, wr2_ref, bl2_ref,
          wrz0_ref, wn0_ref, whh0_ref, bih0_ref, bhh0_ref,
          wrz1_ref, wn1_ref, whh1_ref, bih1_ref, bhh1_ref,
          wfc_ref, fcb_ref, out_ref, seqa_ref, seqb_ref):
    f32 = jnp.float32
    bf16 = jnp.bfloat16

    def ghh(hc, whh_ref, bhh_ref):
        return jnp.dot(hc.astype(bf16), whh_ref[:],
                       preferred_element_type=f32) + bhh_ref[:]

    def gih(x, wrz_ref, wn_ref, bih_ref):
        rz = jnp.dot(x.astype(bf16), wrz_ref[:], preferred_element_type=f32)
        n = jnp.dot(x, wn_ref[:], preferred_element_type=f32)
        return rz + bih_ref[:, :2 * H], n + bih_ref[:, 2 * H:]

    w12 = jnp.dot(wr1_ref[:], wr2_ref[:], preferred_element_type=f32)
    b12 = jnp.dot(bl1_ref[:], wr2_ref[:], preferred_element_type=f32) + bl2_ref[:]
    seqa_ref[:] = jnp.dot(x0_ref[:], w12, preferred_element_type=f32) + b12

    h0 = jnp.zeros((B, H), f32)
    h1 = jnp.zeros((B, H), f32)
    out_acc = jnp.zeros((B, OUTP), f32) + fcb_ref[:]
    bufs = (seqa_ref, seqb_ref)

    for k in range(HOR):
        src = bufs[k % 2]
        dst = bufs[(k + 1) % 2]

        def l0_cell(t, hc, _src=src):
            rz, n = gih(_src[pl.ds(t * B, B), :], wrz0_ref, wn0_ref, bih0_ref)
            return _cell(rz, n, ghh(hc, whh0_ref, bhh0_ref), hc)

        def l1_cell(x, hc):
            rz, n = gih(x, wrz1_ref, wn1_ref, bih1_ref)
            return _cell(rz, n, ghh(hc, whh1_ref, bhh1_ref), hc)

        h0 = l0_cell(0, h0)

        def body(t, carry, _l0=l0_cell, _l1=l1_cell, _dst=dst):
            hc0, hc1, h0d = carry
            hn0 = _l0(t, hc0)
            hn1 = _l1(h0d, hc1)
            _dst[pl.ds((t - 1) * B, B), :] = hn1
            return hn0, hn1, hn0

        h0, h1, h0d = jax.lax.fori_loop(1, W, body, (h0, h1, h0))

        h1 = l1_cell(h0d, h1)
        dst[pl.ds((W - 1) * B, B), :] = h1

        out_acc = out_acc + jnp.dot(h1, wfc_ref[k * H:(k + 1) * H, :],
                                    preferred_element_type=f32)
    out_ref[:] = out_acc


def kernel(features, Wl1, bl1, Wr1, Wl2, bl2, Wr2, Wih0, Whh0, bih0, bhh0,
           Wih1, Whh1, bih1, bhh1, fc_w, fc_b):
    f32 = jnp.float32
    bf16 = jnp.bfloat16
    x0 = jnp.transpose(features[:, :, 0, :], (1, 0, 2)).reshape(W * B, H)
    wfc = jnp.kron(jnp.eye(HOR, OUTP, dtype=f32), fc_w.reshape(H, 1))
    fcb = jnp.broadcast_to(fc_b.reshape(1, 1), (1, OUTP))

    out = pl.pallas_call(
        _body,
        out_shape=jax.ShapeDtypeStruct((B, OUTP), f32),
        scratch_shapes=[
            pltpu.VMEM((W * B, H), f32),
            pltpu.VMEM((W * B, H), f32),
        ],
    )(x0, Wr1, bl1.reshape(1, H), Wr2, bl2.reshape(1, H),
      Wih0.T[:, :2 * H].astype(bf16), Wih0.T[:, 2 * H:],
      Whh0.T.astype(bf16), bih0.reshape(1, 3 * H), bhh0.reshape(1, 3 * H),
      Wih1.T[:, :2 * H].astype(bf16), Wih1.T[:, 2 * H:],
      Whh1.T.astype(bf16), bih1.reshape(1, 3 * H), bhh1.reshape(1, 3 * H),
      wfc, fcb)
    return out[:, :HOR]


# speedup vs baseline: 1.4018x; 1.4018x over previous
import jax
import jax.numpy as jnp
from jax.experimental import pallas as pl
from jax.experimental.pallas import tpu as pltpu

H = 128
W = 20
B = 256
HOR = 12
UNROLL = 6
OUTP = 128


def _cell(gi, gh, hc):
    r = jax.nn.sigmoid(gi[:, :H] + gh[:, :H])
    z = jax.nn.sigmoid(gi[:, H:2 * H] + gh[:, H:2 * H])
    n = jnp.tanh(gi[:, 2 * H:] + r * gh[:, 2 * H:])
    return (1.0 - z) * n + z * hc


def _body(x0_ref, wr1_ref, bl1_ref, wr2_ref, bl2_ref,
          wih0_ref, whh0_ref, bih0_ref, bhh0_ref,
          wih1_ref, whh1_ref, bih1_ref, bhh1_ref,
          wfc_ref, fcb_ref, out_ref, seq_ref, gi_ref):
    f32 = jnp.float32
    bf16 = jnp.bfloat16

    def ghh(hc, whh_ref, bhh_ref):
        return jnp.dot(hc.astype(bf16), whh_ref[:],
                       preferred_element_type=f32) + bhh_ref[:]

    def l0_cell(t, hc):
        return _cell(gi_ref[pl.ds(t * B, B), :],
                     ghh(hc, whh0_ref, bhh0_ref), hc)

    def l1_cell(x, hc):
        gi = jnp.dot(x, wih1_ref[:], preferred_element_type=f32) + bih1_ref[:]
        return _cell(gi, ghh(hc, whh1_ref, bhh1_ref), hc)

    w12 = jnp.dot(wr1_ref[:], wr2_ref[:], preferred_element_type=f32)
    b12 = jnp.dot(bl1_ref[:], wr2_ref[:], preferred_element_type=f32) + bl2_ref[:]
    seq_ref[:] = jnp.dot(x0_ref[:], w12, preferred_element_type=f32) + b12

    h0 = jnp.zeros((B, H), f32)
    h1 = jnp.zeros((B, H), f32)
    out_acc = jnp.zeros((B, OUTP), f32) + fcb_ref[:]

    for k in range(HOR):
        gi_ref[:] = (jnp.dot(seq_ref[:], wih0_ref[:], preferred_element_type=f32)
                     + bih0_ref[:])

        h0 = l0_cell(0, h0)

        def body(i, carry):
            hc0, hc1, h0d = carry
            for j in range(UNROLL):
                t = i * UNROLL + (1 + j)
                hn0 = l0_cell(t, hc0)
                hn1 = l1_cell(h0d, hc1)
                seq_ref[pl.ds((t - 1) * B, B), :] = hn1
                hc0, hc1, h0d = hn0, hn1, hn0
            return hc0, hc1, h0d

        h0, h1, h0d = jax.lax.fori_loop(0, (W - 2) // UNROLL, body,
                                        (h0, h1, h0))

        hn0 = l0_cell(W - 1, h0)
        hn1 = l1_cell(h0d, h1)
        seq_ref[pl.ds((W - 2) * B, B), :] = hn1
        h1 = l1_cell(hn0, hn1)
        seq_ref[pl.ds((W - 1) * B, B), :] = h1
        h0 = hn0

        out_acc = out_acc + jnp.dot(h1, wfc_ref[k * H:(k + 1) * H, :],
                                    preferred_element_type=f32)
    out_ref[:] = out_acc


def kernel(features, Wl1, bl1, Wr1, Wl2, bl2, Wr2, Wih0, Whh0, bih0, bhh0,
           Wih1, Whh1, bih1, bhh1, fc_w, fc_b):
    f32 = jnp.float32
    x0 = jnp.transpose(features[:, :, 0, :], (1, 0, 2)).reshape(W * B, H)
    wfc = jnp.kron(jnp.eye(HOR, OUTP, dtype=f32), fc_w.reshape(H, 1))
    fcb = jnp.broadcast_to(fc_b.reshape(1, 1), (1, OUTP))

    out = pl.pallas_call(
        _body,
        out_shape=jax.ShapeDtypeStruct((B, OUTP), f32),
        scratch_shapes=[
            pltpu.VMEM((W * B, H), f32),
            pltpu.VMEM((W * B, 3 * H), f32),
        ],
    )(x0, Wr1, bl1.reshape(1, H), Wr2, bl2.reshape(1, H),
      Wih0.T, Whh0.T.astype(jnp.bfloat16), bih0.reshape(1, 3 * H),
      bhh0.reshape(1, 3 * H),
      Wih1.T, Whh1.T.astype(jnp.bfloat16), bih1.reshape(1, 3 * H),
      bhh1.reshape(1, 3 * H),
      wfc, fcb)
    return out[:, :HOR]


# speedup vs baseline: 1.4175x; 1.0112x over previous
import jax
import jax.numpy as jnp
from jax.experimental import pallas as pl
from jax.experimental.pallas import tpu as pltpu

H = 128
W = 20
B = 256
HOR = 12
UNROLL = 6
OUTP = 128


def _cell(gi, gh, hc):
    r = jax.nn.sigmoid(gi[:, :H] + gh[:, :H])
    z = jax.nn.sigmoid(gi[:, H:2 * H] + gh[:, H:2 * H])
    n = jnp.tanh(gi[:, 2 * H:] + r * gh[:, 2 * H:])
    return (1.0 - z) * n + z * hc


def _body(x0_ref, wr1_ref, bl1_ref, wr2_ref, bl2_ref,
          wih0_ref, whh0_ref, bih0_ref, bhh0_ref,
          wih1_ref, whh1_ref, bih1_ref, bhh1_ref,
          wfc_ref, fcb_ref, out_ref, seq_ref, gi_ref):
    f32 = jnp.float32
    bf16 = jnp.bfloat16

    def ghh(hc, whh_ref, bhh_ref):
        return jnp.dot(hc.astype(bf16), whh_ref[:],
                       preferred_element_type=f32) + bhh_ref[:]

    def l0_cell(t, hc):
        return _cell(gi_ref[pl.ds(t * B, B), :],
                     ghh(hc, whh0_ref, bhh0_ref), hc)

    def l1_cell(x, hc):
        gi = jnp.dot(x, wih1_ref[:], preferred_element_type=f32) + bih1_ref[:]
        return _cell(gi, ghh(hc, whh1_ref, bhh1_ref), hc)

    w12 = jnp.dot(wr1_ref[:], wr2_ref[:], preferred_element_type=f32)
    b12 = jnp.dot(bl1_ref[:], wr2_ref[:], preferred_element_type=f32) + bl2_ref[:]
    seq_ref[:] = jnp.dot(x0_ref[:], w12, preferred_element_type=f32) + b12

    h0 = jnp.zeros((B, H), f32)
    h1 = jnp.zeros((B, H), f32)
    out_acc = jnp.zeros((B, OUTP), f32) + fcb_ref[:]

    def gi0(x):
        return jnp.dot(x, wih0_ref[:], preferred_element_type=f32) + bih0_ref[:]

    gi_ref[:] = gi0(seq_ref[:])

    for k in range(HOR):
        h0 = l0_cell(0, h0)

        def body(i, carry):
            hc0, hc1, h0d = carry
            for j in range(UNROLL):
                t = i * UNROLL + (1 + j)
                hn0 = l0_cell(t, hc0)
                hn1 = l1_cell(h0d, hc1)
                seq_ref[pl.ds((t - 1) * B, B), :] = hn1
                hc0, hc1, h0d = hn0, hn1, hn0
            return hc0, hc1, h0d

        h0, h1, h0d = jax.lax.fori_loop(0, (W - 2) // UNROLL, body,
                                        (h0, h1, h0))

        hn0 = l0_cell(W - 1, h0)
        hn1 = l1_cell(h0d, h1)
        h1 = l1_cell(hn0, hn1)
        h0 = hn0
        if k < HOR - 1:
            gi_ref[pl.ds((W - 2) * B, B), :] = gi0(hn1)
            gi_ref[pl.ds((W - 1) * B, B), :] = gi0(h1)
            gi_ref[pl.ds(0, (W - 2) * B), :] = gi0(
                seq_ref[pl.ds(0, (W - 2) * B), :])

        out_acc = out_acc + jnp.dot(h1, wfc_ref[k * H:(k + 1) * H, :],
                                    preferred_element_type=f32)
    out_ref[:] = out_acc


def kernel(features, Wl1, bl1, Wr1, Wl2, bl2, Wr2, Wih0, Whh0, bih0, bhh0,
           Wih1, Whh1, bih1, bhh1, fc_w, fc_b):
    f32 = jnp.float32
    x0 = jnp.transpose(features[:, :, 0, :], (1, 0, 2)).reshape(W * B, H)
    wfc = jnp.kron(jnp.eye(HOR, OUTP, dtype=f32), fc_w.reshape(H, 1))
    fcb = jnp.broadcast_to(fc_b.reshape(1, 1), (1, OUTP))

    out = pl.pallas_call(
        _body,
        out_shape=jax.ShapeDtypeStruct((B, OUTP), f32),
        scratch_shapes=[
            pltpu.VMEM((W * B, H), f32),
            pltpu.VMEM((W * B, 3 * H), f32),
        ],
    )(x0, Wr1, bl1.reshape(1, H), Wr2, bl2.reshape(1, H),
      Wih0.T, Whh0.T.astype(jnp.bfloat16), bih0.reshape(1, 3 * H),
      bhh0.reshape(1, 3 * H),
      Wih1.T, Whh1.T.astype(jnp.bfloat16), bih1.reshape(1, 3 * H),
      bhh1.reshape(1, 3 * H),
      wfc, fcb)
    return out[:, :HOR]


# speedup vs baseline: 1.4329x; 1.0109x over previous
import jax
import jax.numpy as jnp
from jax.experimental import pallas as pl
from jax.experimental.pallas import tpu as pltpu

H = 128
W = 20
B = 256
HOR = 12
UNROLL = 9
OUTP = 128


def _cell(gi, gh, hc):
    r = jax.nn.sigmoid(gi[:, :H] + gh[:, :H])
    z = jax.nn.sigmoid(gi[:, H:2 * H] + gh[:, H:2 * H])
    n = jnp.tanh(gi[:, 2 * H:] + r * gh[:, 2 * H:])
    return (1.0 - z) * n + z * hc


def _body(x0_ref, wr1_ref, bl1_ref, wr2_ref, bl2_ref,
          wih0_ref, whh0_ref, bih0_ref, bhh0_ref,
          wih1_ref, whh1_ref, bih1_ref, bhh1_ref,
          wfc_ref, fcb_ref, out_ref, seq_ref, gi_ref):
    f32 = jnp.float32
    bf16 = jnp.bfloat16

    def ghh(hc, whh_ref, bhh_ref):
        return jnp.dot(hc.astype(bf16), whh_ref[:],
                       preferred_element_type=f32) + bhh_ref[:]

    def l0_cell(t, hc):
        return _cell(gi_ref[pl.ds(t * B, B), :],
                     ghh(hc, whh0_ref, bhh0_ref), hc)

    def l1_cell(x, hc):
        gi = jnp.dot(x, wih1_ref[:], preferred_element_type=f32) + bih1_ref[:]
        return _cell(gi, ghh(hc, whh1_ref, bhh1_ref), hc)

    w12 = jnp.dot(wr1_ref[:], wr2_ref[:], preferred_element_type=f32)
    b12 = jnp.dot(bl1_ref[:], wr2_ref[:], preferred_element_type=f32) + bl2_ref[:]
    seq_ref[:] = jnp.dot(x0_ref[:], w12, preferred_element_type=f32) + b12

    h0 = jnp.zeros((B, H), f32)
    h1 = jnp.zeros((B, H), f32)
    out_acc = jnp.zeros((B, OUTP), f32) + fcb_ref[:]

    def gi0(x):
        return jnp.dot(x, wih0_ref[:], preferred_element_type=f32) + bih0_ref[:]

    gi_ref[:] = gi0(seq_ref[:])

    for k in range(HOR):
        h0 = l0_cell(0, h0)

        def body(i, carry):
            hc0, hc1, h0d = carry
            for j in range(UNROLL):
                t = i * UNROLL + (1 + j)
                hn0 = l0_cell(t, hc0)
                hn1 = l1_cell(h0d, hc1)
                seq_ref[pl.ds((t - 1) * B, B), :] = hn1
                hc0, hc1, h0d = hn0, hn1, hn0
            return hc0, hc1, h0d

        h0, h1, h0d = jax.lax.fori_loop(0, (W - 2) // UNROLL, body,
                                        (h0, h1, h0))

        hn0 = l0_cell(W - 1, h0)
        hn1 = l1_cell(h0d, h1)
        h1 = l1_cell(hn0, hn1)
        h0 = hn0
        if k < HOR - 1:
            gi_ref[pl.ds((W - 2) * B, B), :] = gi0(hn1)
            gi_ref[pl.ds((W - 1) * B, B), :] = gi0(h1)
            gi_ref[pl.ds(0, (W - 2) * B), :] = gi0(
                seq_ref[pl.ds(0, (W - 2) * B), :])

        out_acc = out_acc + jnp.dot(h1, wfc_ref[k * H:(k + 1) * H, :],
                                    preferred_element_type=f32)
    out_ref[:] = out_acc


def kernel(features, Wl1, bl1, Wr1, Wl2, bl2, Wr2, Wih0, Whh0, bih0, bhh0,
           Wih1, Whh1, bih1, bhh1, fc_w, fc_b):
    f32 = jnp.float32
    x0 = jnp.transpose(features[:, :, 0, :], (1, 0, 2)).reshape(W * B, H)
    wfc = jnp.kron(jnp.eye(HOR, OUTP, dtype=f32), fc_w.reshape(H, 1))
    fcb = jnp.broadcast_to(fc_b.reshape(1, 1), (1, OUTP))

    out = pl.pallas_call(
        _body,
        out_shape=jax.ShapeDtypeStruct((B, OUTP), f32),
        scratch_shapes=[
            pltpu.VMEM((W * B, H), f32),
            pltpu.VMEM((W * B, 3 * H), f32),
        ],
    )(x0, Wr1, bl1.reshape(1, H), Wr2, bl2.reshape(1, H),
      Wih0.T, Whh0.T.astype(jnp.bfloat16), bih0.reshape(1, 3 * H),
      bhh0.reshape(1, 3 * H),
      Wih1.T, Whh1.T.astype(jnp.bfloat16), bih1.reshape(1, 3 * H),
      bhh1.reshape(1, 3 * H),
      wfc, fcb)
    return out[:, :HOR]
